# baseline (speedup 1.0000x reference)
import jax
import jax.numpy as jnp
from jax.experimental import pallas as pl
from jax.experimental.pallas import tpu as pltpu

_VMEM_LIMIT = 64 * 1024 * 1024


def _dsconv_kernel(xm_ref, w_ref, b_ref, o_ref):
    C = xm_ref.shape[1]
    H = xm_ref.shape[2]
    W = xm_ref.shape[3]
    Ho, Wo = H // 2, W // 2
    m = Ho * Wo

    xb = xm_ref[0].astype(jnp.bfloat16)
    xt = jnp.transpose(xb, (1, 2, 0))
    xtp = xt.reshape(Ho, 2, Wo, 2 * C)
    rows0 = xtp[:, 0]
    rows1 = xtp[:, 1]
    rows2 = jnp.concatenate(
        [rows0[1:], jnp.zeros((1, Wo, 2 * C), jnp.bfloat16)], axis=0)

    pieces = []
    for rows in (rows0, rows1, rows2):
        pieces.append(rows.reshape(m, 2 * C))
        s = jnp.concatenate(
            [rows[:, 1:, :C], jnp.zeros((Ho, 1, C), jnp.bfloat16)], axis=1)
        pieces.append(s.reshape(m, C))
    lhs = jnp.concatenate(pieces, axis=-1)

    acc = jax.lax.dot_general(
        w_ref[...], lhs, (((1,), (1,)), ((), ())),
        preferred_element_type=jnp.float32)
    o_ref[0] = (acc + b_ref[...]).reshape(o_ref.shape[1:])


@jax.jit
def kernel(x, weight, bias):
    B, C, H, W = x.shape
    Cout = weight.shape[0]
    Ho, Wo = H // 2, W // 2

    w9 = jnp.transpose(weight, (0, 2, 3, 1)).reshape(Cout, 9 * C)
    w9 = w9.astype(jnp.bfloat16)
    b_col = bias.reshape(Cout, 1).astype(jnp.float32)

    m, k = B * Ho * Wo, 9 * C
    cost = pl.CostEstimate(
        flops=int(2 * m * k * Cout),
        transcendentals=0,
        bytes_accessed=int(x.size * 4 + w9.size * 2 + m * Cout * 4))

    out = pl.pallas_call(
        _dsconv_kernel,
        out_shape=jax.ShapeDtypeStruct((B, Cout, Ho, Wo), jnp.float32),
        grid_spec=pltpu.PrefetchScalarGridSpec(
            num_scalar_prefetch=0,
            grid=(B,),
            in_specs=[
                pl.BlockSpec((1, C, H, W), lambda b: (b, 0, 0, 0)),
                pl.BlockSpec((Cout, 9 * C), lambda b: (0, 0)),
                pl.BlockSpec((Cout, 1), lambda b: (0, 0)),
            ],
            out_specs=pl.BlockSpec((1, Cout, Ho, Wo), lambda b: (b, 0, 0, 0)),
        ),
        compiler_params=pltpu.CompilerParams(
            dimension_semantics=("parallel",),
            vmem_limit_bytes=_VMEM_LIMIT),
        cost_estimate=cost,
    )(x, w9, b_col)

    return out


# speedup vs baseline: 1.8555x; 1.8555x over previous
import jax
import jax.numpy as jnp
from jax.experimental import pallas as pl
from jax.experimental.pallas import tpu as pltpu

_VMEM_LIMIT = 64 * 1024 * 1024


def _dsconv_kernel(xm_ref, w_ref, b_ref, o_ref):
    C = xm_ref.shape[1]
    H = xm_ref.shape[2]
    W = xm_ref.shape[3]
    Ho, Wo = H // 2, W // 2
    m = Ho * Wo

    xb = xm_ref[0].astype(jnp.bfloat16)
    xt = jnp.transpose(xb, (1, 2, 0))
    xtp = xt.reshape(Ho, 2, Wo, 2 * C)
    rows0 = xtp[:, 0]
    rows1 = xtp[:, 1]
    rows2 = jnp.concatenate(
        [rows0[1:], jnp.zeros((1, Wo, 2 * C), jnp.bfloat16)], axis=0)

    pieces = []
    for rows in (rows0, rows1, rows2):
        pieces.append(rows.reshape(m, 2 * C))
        s = jnp.concatenate(
            [rows[:, 1:, :C], jnp.zeros((Ho, 1, C), jnp.bfloat16)], axis=1)
        pieces.append(s.reshape(m, C))
    lhs = jnp.concatenate(pieces, axis=-1)

    acc = jnp.dot(lhs, w_ref[...],
                  preferred_element_type=jnp.float32)
    o_ref[0] = (acc + b_ref[...]).reshape(o_ref.shape[1:])


@jax.jit
def kernel(x, weight, bias):
    B, C, H, W = x.shape
    Cout = weight.shape[0]
    Ho, Wo = H // 2, W // 2

    w9 = jnp.transpose(weight, (2, 3, 1, 0)).reshape(9 * C, Cout)
    w9 = w9.astype(jnp.bfloat16)
    b_row = bias.reshape(1, Cout).astype(jnp.float32)

    m, k = B * Ho * Wo, 9 * C
    cost = pl.CostEstimate(
        flops=int(2 * m * k * Cout),
        transcendentals=0,
        bytes_accessed=int(x.size * 4 + w9.size * 2 + m * Cout * 4))

    out = pl.pallas_call(
        _dsconv_kernel,
        out_shape=jax.ShapeDtypeStruct((B, Ho, Wo, Cout), jnp.float32),
        grid_spec=pltpu.PrefetchScalarGridSpec(
            num_scalar_prefetch=0,
            grid=(B,),
            in_specs=[
                pl.BlockSpec((1, C, H, W), lambda b: (b, 0, 0, 0)),
                pl.BlockSpec((9 * C, Cout), lambda b: (0, 0)),
                pl.BlockSpec((1, Cout), lambda b: (0, 0)),
            ],
            out_specs=pl.BlockSpec((1, Ho, Wo, Cout), lambda b: (b, 0, 0, 0)),
        ),
        compiler_params=pltpu.CompilerParams(
            dimension_semantics=("parallel",),
            vmem_limit_bytes=_VMEM_LIMIT),
        cost_estimate=cost,
    )(x, w9, b_row)

    return jnp.transpose(out, (0, 3, 1, 2))
